# baseline (speedup 1.0000x reference)
import functools

import jax
import jax.numpy as jnp
from jax import lax
from jax.experimental import pallas as pl
from jax.experimental.pallas import tpu as pltpu
from jax.experimental.pallas import tpu_sc as plsc

IN_F = 2048
HID = 8192
OUT_F = 2048
N_TOK = 4096
K_SEL = 2048

NC = 2
NS = 16
NW = NC * NS
ROWS_PER_W = K_SEL // NW
CHUNK = 16
NCHUNK = ROWS_PER_W // CHUNK


def _gather_rows_pipelined(tbl_hbm, out_hbm, base, idx_v, bufs, gsems, osems):
    gathers = [None, None]
    outs = [None, None]
    for t in range(NCHUNK):
        b = t % 2
        if outs[b] is not None:
            outs[b].wait()
        idxs = idx_v[pl.ds(t * CHUNK, CHUNK)]
        gathers[b] = pltpu.async_copy(tbl_hbm.at[idxs], bufs[b], gsems[b])
        if t > 0:
            pb = (t - 1) % 2
            gathers[pb].wait()
            outs[pb] = pltpu.async_copy(
                bufs[pb], out_hbm.at[pl.ds(base + (t - 1) * CHUNK, CHUNK)],
                osems[pb])
    lb = (NCHUNK - 1) % 2
    gathers[lb].wait()
    outs[lb] = pltpu.async_copy(
        bufs[lb], out_hbm.at[pl.ds(base + (NCHUNK - 1) * CHUNK, CHUNK)],
        osems[lb])
    outs[0].wait()
    outs[1].wait()


@functools.cache
def _get_sc_gather_w1b1():
    mesh = plsc.VectorSubcoreMesh(core_axis_name="c", subcore_axis_name="s",
                                  num_cores=NC, num_subcores=NS)

    @functools.partial(
        pl.kernel,
        out_type=(
            jax.ShapeDtypeStruct((K_SEL, IN_F), jnp.float32),
            jax.ShapeDtypeStruct((K_SEL,), jnp.float32),
        ),
        mesh=mesh,
        compiler_params=pltpu.CompilerParams(needs_layout_passes=False),
        scratch_types=[
            pltpu.VMEM((ROWS_PER_W,), jnp.int32),
            pltpu.VMEM((CHUNK, IN_F), jnp.float32),
            pltpu.VMEM((CHUNK, IN_F), jnp.float32),
            pltpu.VMEM((HID,), jnp.float32),
            pltpu.VMEM((ROWS_PER_W,), jnp.float32),
            pltpu.SemaphoreType.DMA,
            pltpu.SemaphoreType.DMA,
            pltpu.SemaphoreType.DMA,
            pltpu.SemaphoreType.DMA,
        ],
    )
    def _sc_gather(fc1_w_hbm, fc1_b_hbm, idx_hbm,
                   w1_out, b1_out,
                   idx_v, bufa, bufb, bias_v, bsel_v,
                   gsem_a, gsem_b, osem_a, osem_b):
        wid = lax.axis_index("s") * NC + lax.axis_index("c")
        base = wid * ROWS_PER_W
        pltpu.sync_copy(idx_hbm.at[pl.ds(base, ROWS_PER_W)], idx_v)

        pltpu.sync_copy(fc1_b_hbm, bias_v)
        for c in range(ROWS_PER_W // 16):
            idxs = idx_v[pl.ds(c * 16, 16)]
            bsel_v[pl.ds(c * 16, 16)] = plsc.load_gather(bias_v, [idxs])
        pltpu.sync_copy(bsel_v, b1_out.at[pl.ds(base, ROWS_PER_W)])

        _gather_rows_pipelined(fc1_w_hbm, w1_out, base, idx_v, (bufa, bufb),
                               (gsem_a, gsem_b), (osem_a, osem_b))

    return _sc_gather


@functools.cache
def _get_sc_gather_w2():
    mesh = plsc.VectorSubcoreMesh(core_axis_name="c", subcore_axis_name="s",
                                  num_cores=NC, num_subcores=NS)

    @functools.partial(
        pl.kernel,
        out_type=jax.ShapeDtypeStruct((K_SEL, OUT_F), jnp.float32),
        mesh=mesh,
        compiler_params=pltpu.CompilerParams(needs_layout_passes=False),
        scratch_types=[
            pltpu.VMEM((ROWS_PER_W,), jnp.int32),
            pltpu.VMEM((CHUNK, OUT_F), jnp.float32),
            pltpu.VMEM((CHUNK, OUT_F), jnp.float32),
            pltpu.SemaphoreType.DMA,
            pltpu.SemaphoreType.DMA,
            pltpu.SemaphoreType.DMA,
            pltpu.SemaphoreType.DMA,
        ],
    )
    def _sc_gather(fc2_w_hbm, idx_hbm, w2_out,
                   idx_v, bufa, bufb,
                   gsem_a, gsem_b, osem_a, osem_b):
        wid = lax.axis_index("s") * NC + lax.axis_index("c")
        base = wid * ROWS_PER_W
        pltpu.sync_copy(idx_hbm.at[pl.ds(base, ROWS_PER_W)], idx_v)
        _gather_rows_pipelined(fc2_w_hbm, w2_out, base, idx_v, (bufa, bufb),
                               (gsem_a, gsem_b), (osem_a, osem_b))

    return _sc_gather


BM = 512
STAGE = 256
NSTAGE = K_SEL // STAGE


def _stage_weights(src, dst, stgs, sems):
    def _sl(s, b):
        return pltpu.make_async_copy(
            src.at[pl.ds(s * STAGE, STAGE), :], stgs[b], sems[b])
    for s in range(NSTAGE):
        b = s % 2
        _sl(s, b).start()
        if s > 0:
            pb = 1 - b
            _sl(s - 1, pb).wait()
            dst[pl.ds((s - 1) * STAGE, STAGE), :] = (
                stgs[pb][...].astype(jnp.bfloat16))
    lb = (NSTAGE - 1) % 2
    _sl(NSTAGE - 1, lb).wait()
    dst[pl.ds((NSTAGE - 1) * STAGE, STAGE), :] = (
        stgs[lb][...].astype(jnp.bfloat16))


def _mlp1_body(x_ref, w1_ref, b1_ref, o_ref, w1bf, stg_a, stg_b,
               sem_a, sem_b):
    @pl.when(pl.program_id(0) == 0)
    def _init():
        _stage_weights(w1_ref, w1bf, (stg_a, stg_b), (sem_a, sem_b))

    xb = x_ref[...].astype(jnp.bfloat16)
    h = lax.dot_general(xb, w1bf[...], (((1,), (1,)), ((), ())),
                        preferred_element_type=jnp.float32)
    o_ref[...] = jnp.maximum(h + b1_ref[...], 0.0).astype(jnp.bfloat16)


_mlp1 = pl.pallas_call(
    _mlp1_body,
    grid=(N_TOK // BM,),
    in_specs=[
        pl.BlockSpec((BM, IN_F), lambda i: (i, 0)),
        pl.BlockSpec(memory_space=pl.ANY),
        pl.BlockSpec((1, K_SEL), lambda i: (0, 0)),
    ],
    out_specs=pl.BlockSpec((BM, K_SEL), lambda i: (i, 0)),
    out_shape=jax.ShapeDtypeStruct((N_TOK, K_SEL), jnp.bfloat16),
    scratch_shapes=[
        pltpu.VMEM((K_SEL, IN_F), jnp.bfloat16),
        pltpu.VMEM((STAGE, IN_F), jnp.float32),
        pltpu.VMEM((STAGE, IN_F), jnp.float32),
        pltpu.SemaphoreType.DMA,
        pltpu.SemaphoreType.DMA,
    ],
    compiler_params=pltpu.CompilerParams(
        dimension_semantics=("arbitrary",),
        vmem_limit_bytes=120 * 1024 * 1024,
    ),
)


def _mlp2_body(h_ref, w2_ref, b2_ref, o_ref, w2bf, stg_a, stg_b,
               sem_a, sem_b):
    @pl.when(pl.program_id(0) == 0)
    def _init():
        _stage_weights(w2_ref, w2bf, (stg_a, stg_b), (sem_a, sem_b))

    y = lax.dot_general(h_ref[...], w2bf[...], (((1,), (0,)), ((), ())),
                        preferred_element_type=jnp.float32)
    o_ref[...] = y + b2_ref[...]


_mlp2 = pl.pallas_call(
    _mlp2_body,
    grid=(N_TOK // BM,),
    in_specs=[
        pl.BlockSpec((BM, K_SEL), lambda i: (i, 0)),
        pl.BlockSpec(memory_space=pl.ANY),
        pl.BlockSpec((1, OUT_F), lambda i: (0, 0)),
    ],
    out_specs=pl.BlockSpec((BM, OUT_F), lambda i: (i, 0)),
    out_shape=jax.ShapeDtypeStruct((N_TOK, OUT_F), jnp.float32),
    scratch_shapes=[
        pltpu.VMEM((K_SEL, OUT_F), jnp.bfloat16),
        pltpu.VMEM((STAGE, OUT_F), jnp.float32),
        pltpu.VMEM((STAGE, OUT_F), jnp.float32),
        pltpu.SemaphoreType.DMA,
        pltpu.SemaphoreType.DMA,
    ],
    compiler_params=pltpu.CompilerParams(
        dimension_semantics=("arbitrary",),
        vmem_limit_bytes=120 * 1024 * 1024,
    ),
)


def kernel(x, index_vec, fc1_w, fc1_b, fc2_w_t, fc2_b):
    idx = index_vec.astype(jnp.int32)
    w1_sel, b1_sel = _get_sc_gather_w1b1()(fc1_w, fc1_b, idx)
    w2_sel = _get_sc_gather_w2()(fc2_w_t, idx)
    h = _mlp1(x, w1_sel, b1_sel.reshape(1, K_SEL))
    return _mlp2(h, w2_sel, fc2_b.reshape(1, OUT_F))


# speedup vs baseline: 1.0390x; 1.0390x over previous
import functools

import jax
import jax.numpy as jnp
from jax import lax
from jax.experimental import pallas as pl
from jax.experimental.pallas import tpu as pltpu
from jax.experimental.pallas import tpu_sc as plsc

IN_F = 2048
HID = 8192
OUT_F = 2048
N_TOK = 4096
K_SEL = 2048

NC = 2
NS = 16
NW = NC * NS
ROWS_PER_W = K_SEL // NW
CHUNK = 16
NCHUNK = ROWS_PER_W // CHUNK


def _gather_rows_pipelined(tbl_hbm, out_hbm, base, idx_v, bufs, gsems, osems):
    gathers = [None, None]
    outs = [None, None]
    for t in range(NCHUNK):
        b = t % 2
        if outs[b] is not None:
            outs[b].wait()
        idxs = idx_v[pl.ds(t * CHUNK, CHUNK)]
        gathers[b] = pltpu.async_copy(tbl_hbm.at[idxs], bufs[b], gsems[b])
        if t > 0:
            pb = (t - 1) % 2
            gathers[pb].wait()
            outs[pb] = pltpu.async_copy(
                bufs[pb], out_hbm.at[pl.ds(base + (t - 1) * CHUNK, CHUNK)],
                osems[pb])
    lb = (NCHUNK - 1) % 2
    gathers[lb].wait()
    outs[lb] = pltpu.async_copy(
        bufs[lb], out_hbm.at[pl.ds(base + (NCHUNK - 1) * CHUNK, CHUNK)],
        osems[lb])
    outs[0].wait()
    outs[1].wait()


@functools.cache
def _get_sc_gather_w1b1():
    mesh = plsc.VectorSubcoreMesh(core_axis_name="c", subcore_axis_name="s",
                                  num_cores=NC, num_subcores=NS)

    @functools.partial(
        pl.kernel,
        out_type=(
            jax.ShapeDtypeStruct((K_SEL, IN_F), jnp.float32),
            jax.ShapeDtypeStruct((K_SEL,), jnp.float32),
        ),
        mesh=mesh,
        compiler_params=pltpu.CompilerParams(needs_layout_passes=False),
        scratch_types=[
            pltpu.VMEM((ROWS_PER_W,), jnp.int32),
            pltpu.VMEM((CHUNK, IN_F), jnp.float32),
            pltpu.VMEM((CHUNK, IN_F), jnp.float32),
            pltpu.VMEM((HID,), jnp.float32),
            pltpu.VMEM((ROWS_PER_W,), jnp.float32),
            pltpu.SemaphoreType.DMA,
            pltpu.SemaphoreType.DMA,
            pltpu.SemaphoreType.DMA,
            pltpu.SemaphoreType.DMA,
        ],
    )
    def _sc_gather(fc1_w_hbm, fc1_b_hbm, idx_hbm,
                   w1_out, b1_out,
                   idx_v, bufa, bufb, bias_v, bsel_v,
                   gsem_a, gsem_b, osem_a, osem_b):
        wid = lax.axis_index("s") * NC + lax.axis_index("c")
        base = wid * ROWS_PER_W
        pltpu.sync_copy(idx_hbm.at[pl.ds(base, ROWS_PER_W)], idx_v)

        pltpu.sync_copy(fc1_b_hbm, bias_v)
        for c in range(ROWS_PER_W // 16):
            idxs = idx_v[pl.ds(c * 16, 16)]
            bsel_v[pl.ds(c * 16, 16)] = plsc.load_gather(bias_v, [idxs])
        pltpu.sync_copy(bsel_v, b1_out.at[pl.ds(base, ROWS_PER_W)])

        _gather_rows_pipelined(fc1_w_hbm, w1_out, base, idx_v, (bufa, bufb),
                               (gsem_a, gsem_b), (osem_a, osem_b))

    return _sc_gather


@functools.cache
def _get_sc_gather_w2():
    mesh = plsc.VectorSubcoreMesh(core_axis_name="c", subcore_axis_name="s",
                                  num_cores=NC, num_subcores=NS)

    @functools.partial(
        pl.kernel,
        out_type=jax.ShapeDtypeStruct((K_SEL, OUT_F), jnp.float32),
        mesh=mesh,
        compiler_params=pltpu.CompilerParams(needs_layout_passes=False),
        scratch_types=[
            pltpu.VMEM((ROWS_PER_W,), jnp.int32),
            pltpu.VMEM((CHUNK, OUT_F), jnp.float32),
            pltpu.VMEM((CHUNK, OUT_F), jnp.float32),
            pltpu.SemaphoreType.DMA,
            pltpu.SemaphoreType.DMA,
            pltpu.SemaphoreType.DMA,
            pltpu.SemaphoreType.DMA,
        ],
    )
    def _sc_gather(fc2_w_hbm, idx_hbm, w2_out,
                   idx_v, bufa, bufb,
                   gsem_a, gsem_b, osem_a, osem_b):
        wid = lax.axis_index("s") * NC + lax.axis_index("c")
        base = wid * ROWS_PER_W
        pltpu.sync_copy(idx_hbm.at[pl.ds(base, ROWS_PER_W)], idx_v)
        _gather_rows_pipelined(fc2_w_hbm, w2_out, base, idx_v, (bufa, bufb),
                               (gsem_a, gsem_b), (osem_a, osem_b))

    return _sc_gather


BM = 512


def _mlp1_body(x_ref, w1_ref, b1_ref, o_ref, w1bf):
    @pl.when(pl.program_id(0) == 0)
    def _init():
        w1bf[...] = w1_ref[...].astype(jnp.bfloat16)

    xb = x_ref[...].astype(jnp.bfloat16)
    h = lax.dot_general(xb, w1bf[...], (((1,), (1,)), ((), ())),
                        preferred_element_type=jnp.float32)
    o_ref[...] = jnp.maximum(h + b1_ref[...], 0.0).astype(jnp.bfloat16)


_mlp1 = pl.pallas_call(
    _mlp1_body,
    grid=(N_TOK // BM,),
    in_specs=[
        pl.BlockSpec((BM, IN_F), lambda i: (i, 0)),
        pl.BlockSpec((K_SEL, IN_F), lambda i: (0, 0)),
        pl.BlockSpec((1, K_SEL), lambda i: (0, 0)),
    ],
    out_specs=pl.BlockSpec((BM, K_SEL), lambda i: (i, 0)),
    out_shape=jax.ShapeDtypeStruct((N_TOK, K_SEL), jnp.bfloat16),
    scratch_shapes=[pltpu.VMEM((K_SEL, IN_F), jnp.bfloat16)],
    compiler_params=pltpu.CompilerParams(
        dimension_semantics=("arbitrary",),
        vmem_limit_bytes=120 * 1024 * 1024,
    ),
)


def _mlp2_body(h_ref, w2_ref, b2_ref, o_ref, w2bf):
    @pl.when(pl.program_id(0) == 0)
    def _init():
        w2bf[...] = w2_ref[...].astype(jnp.bfloat16)

    y = lax.dot_general(h_ref[...], w2bf[...], (((1,), (0,)), ((), ())),
                        preferred_element_type=jnp.float32)
    o_ref[...] = y + b2_ref[...]


_mlp2 = pl.pallas_call(
    _mlp2_body,
    grid=(N_TOK // BM,),
    in_specs=[
        pl.BlockSpec((BM, K_SEL), lambda i: (i, 0)),
        pl.BlockSpec((K_SEL, OUT_F), lambda i: (0, 0)),
        pl.BlockSpec((1, OUT_F), lambda i: (0, 0)),
    ],
    out_specs=pl.BlockSpec((BM, OUT_F), lambda i: (i, 0)),
    out_shape=jax.ShapeDtypeStruct((N_TOK, OUT_F), jnp.float32),
    scratch_shapes=[pltpu.VMEM((K_SEL, OUT_F), jnp.bfloat16)],
    compiler_params=pltpu.CompilerParams(
        dimension_semantics=("arbitrary",),
        vmem_limit_bytes=120 * 1024 * 1024,
    ),
)


def kernel(x, index_vec, fc1_w, fc1_b, fc2_w_t, fc2_b):
    idx = index_vec.astype(jnp.int32)
    w1_sel, b1_sel = _get_sc_gather_w1b1()(fc1_w, fc1_b, idx)
    w2_sel = _get_sc_gather_w2()(fc2_w_t, idx)
    h = _mlp1(x, w1_sel, b1_sel.reshape(1, K_SEL))
    return _mlp2(h, w2_sel, fc2_b.reshape(1, OUT_F))
